# Initial kernel scaffold; baseline (speedup 1.0000x reference)
import functools

import jax
import jax.numpy as jnp
from jax import lax
from jax.experimental import pallas as pl
from jax.experimental.pallas import tpu as pltpu
from jax.experimental.pallas import tpu_sc as plsc

N = 10000
E = 320000
NW = 32
EPW = E // NW
K = 80
NCH = EPW // K
NRC = N // K
RW1 = 144
RW3 = 48


def _make_mp(RW, heads):
    NV = RW // 16
    mesh = plsc.VectorSubcoreMesh(core_axis_name="c", subcore_axis_name="s")

    def body(tableA, tableB, srcr, dstr, out, src_v, dst_v, bufA, bufB,
             stage, acc, semA, semB):
        cid = lax.axis_index("c")
        sid = lax.axis_index("s")
        wid = sid * 2 + cid

        pltpu.sync_copy(srcr.at[wid], src_v)
        pltpu.sync_copy(dstr.at[wid], dst_v)

        zero16 = jnp.zeros((16,), jnp.float32)

        def zrow(r, carry):
            for v in range(NV):
                stage[r, pl.ds(16 * v, 16)] = zero16
            return carry

        lax.fori_loop(0, K, zrow, 0)

        def zchunk(i, carry):
            ch = sid + 16 * i

            @pl.when(ch < NRC)
            def _():
                pltpu.sync_copy(stage, acc.at[pl.ds(pl.multiple_of(ch * K, 8), K)])
            return carry

        lax.fori_loop(0, (NRC + 15) // 16, zchunk, 0)
        plsc.subcore_barrier()

        shift_idx = (lax.iota(jnp.int32, 16) & 7) + 8
        full8 = jnp.full((16,), 8, jnp.int32)
        full0 = jnp.full((16,), 0, jnp.int32)

        def chunk(j, carry):
            cpA = pltpu.async_copy(tableA.at[src_v.at[j]], bufA, semA)
            cpB = pltpu.async_copy(tableB.at[dst_v.at[j]], bufB, semB)
            cpA.wait()
            cpB.wait()

            def edge(e, c2):
                vb = bufB[e, pl.ds(0, 16)]
                if heads == 8:
                    t = bufA[e, pl.ds(128, 16)]
                    va = jnp.take(t, shift_idx, mode="promise_in_bounds")
                else:
                    t = bufA[e, pl.ds(32, 16)]
                    va = jnp.take(t, full8, mode="promise_in_bounds")
                    vb = jnp.take(vb, full0, mode="promise_in_bounds")
                sv = va + vb
                exv = jnp.exp(jnp.maximum(sv, 0.2 * sv))
                if heads == 8:
                    for h in range(8):
                        exh = jnp.take(exv, jnp.full((16,), h, jnp.int32),
                                       mode="promise_in_bounds")
                        stage[e, pl.ds(16 * h, 16)] = (
                            bufA[e, pl.ds(16 * h, 16)] * exh)
                    stage[e, pl.ds(128, 16)] = t * exv
                else:
                    for v in range(NV):
                        stage[e, pl.ds(16 * v, 16)] = (
                            bufA[e, pl.ds(16 * v, 16)] * exv)
                return c2

            lax.fori_loop(0, K, edge, 0)
            pltpu.sync_copy(stage, acc.at[dst_v.at[j]], add=True)
            return carry

        lax.fori_loop(0, NCH, chunk, 0)
        plsc.subcore_barrier()

        def wchunk(i, carry):
            ch = sid + 16 * i

            @pl.when(ch < NRC)
            def _():
                st = pl.multiple_of(ch * K, 8)
                pltpu.sync_copy(acc.at[pl.ds(st, K)], out.at[cid, pl.ds(st, K)])
            return carry

        lax.fori_loop(0, (NRC + 15) // 16, wchunk, 0)

    return pl.kernel(
        body,
        mesh=mesh,
        out_type=jax.ShapeDtypeStruct((2, N, RW), jnp.float32),
        scratch_types=[
            pltpu.VMEM((NCH, K), jnp.int32),
            pltpu.VMEM((NCH, K), jnp.int32),
            pltpu.VMEM((K, RW), jnp.float32),
            pltpu.VMEM((K, 16), jnp.float32),
            pltpu.VMEM((K, RW), jnp.float32),
            pltpu.VMEM_SHARED((N, RW), jnp.float32),
            pltpu.SemaphoreType.DMA,
            pltpu.SemaphoreType.DMA,
        ],
    )


def _tc_first(x, W, asrc_m, adst_m):
    def f(x_ref, w_ref, as_ref, ad_ref, a_ref, b_ref):
        h = jnp.dot(x_ref[...], w_ref[...], preferred_element_type=jnp.float32)
        a_ref[:, 0:128] = h
        a_ref[:, 128:136] = jnp.ones((N, 8), jnp.float32)
        a_ref[:, 136:144] = jnp.dot(h, as_ref[...],
                                    preferred_element_type=jnp.float32)
        b_ref[:, 0:8] = jnp.dot(h, ad_ref[...],
                                preferred_element_type=jnp.float32)
        b_ref[:, 8:16] = jnp.zeros((N, 8), jnp.float32)

    return pl.pallas_call(
        f,
        out_shape=(jax.ShapeDtypeStruct((N, RW1), jnp.float32),
                   jax.ShapeDtypeStruct((N, 16), jnp.float32)),
    )(x, W, asrc_m, adst_m)


def _combine_bn_relu(p_ref, sx_ref, bias_ref, g_ref, be_ref):
    acc = p_ref[0] + p_ref[1]
    s = jnp.dot(acc[:, 128:136], sx_ref[...],
                preferred_element_type=jnp.float32)
    g = acc[:, 0:128] / (s + 1e-16) + bias_ref[...]
    mu = jnp.mean(g, axis=0, keepdims=True)
    var = jnp.mean((g - mu) ** 2, axis=0, keepdims=True)
    y = g_ref[...] * (g - mu) * lax.rsqrt(var + 1e-5) + be_ref[...]
    return jnp.maximum(y, 0.0)


def _tc_mid(part, gamma, beta, bias, W, asrc_m, adst_m, sexp):
    def f(p_ref, sx_ref, bias_ref, g_ref, be_ref, w_ref, as_ref, ad_ref,
          a_ref, b_ref):
        y = _combine_bn_relu(p_ref, sx_ref, bias_ref, g_ref, be_ref)
        h = jnp.dot(y, w_ref[...], preferred_element_type=jnp.float32)
        a_ref[:, 0:128] = h
        a_ref[:, 128:136] = jnp.ones((N, 8), jnp.float32)
        a_ref[:, 136:144] = jnp.dot(h, as_ref[...],
                                    preferred_element_type=jnp.float32)
        b_ref[:, 0:8] = jnp.dot(h, ad_ref[...],
                                preferred_element_type=jnp.float32)
        b_ref[:, 8:16] = jnp.zeros((N, 8), jnp.float32)

    return pl.pallas_call(
        f,
        out_shape=(jax.ShapeDtypeStruct((N, RW1), jnp.float32),
                   jax.ShapeDtypeStruct((N, 16), jnp.float32)),
    )(part, sexp, bias, gamma, beta, W, asrc_m, adst_m)


def _tc_mid3(part, gamma, beta, bias, W, as3, ad3, sexp):
    def f(p_ref, sx_ref, bias_ref, g_ref, be_ref, w_ref, as_ref, ad_ref,
          a_ref, b_ref):
        y = _combine_bn_relu(p_ref, sx_ref, bias_ref, g_ref, be_ref)
        h = jnp.dot(y, w_ref[...], preferred_element_type=jnp.float32)
        a_ref[:, 0:40] = h
        a_ref[:, 40:41] = jnp.dot(h, as_ref[...],
                                  preferred_element_type=jnp.float32)
        a_ref[:, 41:42] = jnp.ones((N, 1), jnp.float32)
        a_ref[:, 42:48] = jnp.zeros((N, 6), jnp.float32)
        b_ref[:, 0:1] = jnp.dot(h, ad_ref[...],
                                preferred_element_type=jnp.float32)
        b_ref[:, 1:16] = jnp.zeros((N, 15), jnp.float32)

    return pl.pallas_call(
        f,
        out_shape=(jax.ShapeDtypeStruct((N, RW3), jnp.float32),
                   jax.ShapeDtypeStruct((N, 16), jnp.float32)),
    )(part, sexp, bias, gamma, beta, W, as3, ad3)


def _tc_final(part, bias):
    def f(p_ref, bias_ref, o_ref):
        acc = p_ref[0] + p_ref[1]
        logits = acc[:, 0:40] / (acc[:, 41:42] + 1e-16) + bias_ref[...]
        m = jnp.max(logits, axis=1, keepdims=True)
        z = logits - m
        lse = jnp.log(jnp.sum(jnp.exp(z), axis=1, keepdims=True))
        o_ref[...] = z - lse

    return pl.pallas_call(
        f,
        out_shape=jax.ShapeDtypeStruct((N, 40), jnp.float32),
    )(part, bias)


def kernel(x, edge_index, W1, att_src1, att_dst1, bias1, gamma1, beta1,
           W2, att_src2, att_dst2, bias2, gamma2, beta2,
           W3, att_src3, att_dst3, bias3):
    ei = edge_index.astype(jnp.int32)
    srcr = ei[0].reshape(NW, NCH, K)
    dstr = ei[1].reshape(NW, NCH, K)

    mask8 = jnp.kron(jnp.eye(8, dtype=jnp.float32),
                     jnp.ones((16, 1), jnp.float32))
    sexp = jnp.kron(jnp.eye(8, dtype=jnp.float32),
                    jnp.ones((1, 16), jnp.float32))
    asrc1_m = mask8 * att_src1.reshape(128, 1)
    adst1_m = mask8 * att_dst1.reshape(128, 1)
    asrc2_m = mask8 * att_src2.reshape(128, 1)
    adst2_m = mask8 * att_dst2.reshape(128, 1)
    as3 = att_src3.reshape(40, 1)
    ad3 = att_dst3.reshape(40, 1)

    b1 = bias1.reshape(1, 128)
    g1 = gamma1.reshape(1, 128)
    be1 = beta1.reshape(1, 128)
    b2 = bias2.reshape(1, 128)
    g2 = gamma2.reshape(1, 128)
    be2 = beta2.reshape(1, 128)
    b3 = bias3.reshape(1, 40)

    mp144 = _make_mp(RW1, 8)
    mp48 = _make_mp(RW3, 1)

    tA, tB = _tc_first(x, W1, asrc1_m, adst1_m)
    part = mp144(tA, tB, srcr, dstr)
    tA, tB = _tc_mid(part, g1, be1, b1, W2, asrc2_m, adst2_m, sexp)
    part = mp144(tA, tB, srcr, dstr)
    tA, tB = _tc_mid3(part, g2, be2, b2, W3, as3, ad3, sexp)
    part3 = mp48(tA, tB, srcr, dstr)
    return _tc_final(part3, b3)


# speedup vs baseline: 49.9389x; 49.9389x over previous
import jax
import jax.numpy as jnp
from jax import lax
from jax.experimental import pallas as pl
from jax.experimental.pallas import tpu as pltpu
from jax.experimental.pallas import tpu_sc as plsc

N = 10000
E = 320000
K = 80
NRC = N // K
RW12 = 80
RW3 = 48
NCH12 = E // 16 // K
NCH3 = E // 32 // K

_SC_PARAMS = pltpu.CompilerParams(use_tc_tiling_on_sc=False)


def _make_mp12():
    mesh = plsc.VectorSubcoreMesh(core_axis_name="c", subcore_axis_name="s")

    def body(tableA, tableB, srcr, dstr, out, src_v, dst_v, bufA, bufB,
             stage, acc, semA, semB):
        cid = lax.axis_index("c")
        sid = lax.axis_index("s")

        pltpu.sync_copy(srcr.at[cid, sid], src_v)
        pltpu.sync_copy(dstr.at[sid], dst_v)

        zero16 = jnp.zeros((16,), jnp.float32)

        def zrow(r, carry):
            for v in range(5):
                stage[r, pl.ds(16 * v, 16)] = zero16
            return carry

        lax.fori_loop(0, K, zrow, 0)

        def zchunk(i, carry):
            ch = sid + 16 * i

            @pl.when(ch < NRC)
            def _():
                pltpu.sync_copy(stage,
                                acc.at[pl.ds(pl.multiple_of(ch * K, 8), K)])
            return carry

        lax.fori_loop(0, (NRC + 15) // 16, zchunk, 0)
        plsc.subcore_barrier()

        lane4 = lax.iota(jnp.int32, 16) & 3
        shiftA = lane4 + 8
        shiftB = lane4 + 4 * cid

        def chunk(j, carry):
            cpA = pltpu.async_copy(tableA.at[src_v.at[j]], bufA, semA)
            cpB = pltpu.async_copy(tableB.at[dst_v.at[j]], bufB, semB)
            cpA.wait()
            cpB.wait()

            def edge(e, c2):
                t = bufA[e, pl.ds(64, 16)]
                va = t.at[shiftA].get(mode="promise_in_bounds")
                vb = bufB[e, pl.ds(0, 16)]
                vb = vb.at[shiftB].get(mode="promise_in_bounds")
                sv = va + vb
                exv = jnp.exp(jnp.maximum(sv, 0.2 * sv))
                for h in range(4):
                    exh = exv.at[jnp.full((16,), h, jnp.int32)].get(
                        mode="promise_in_bounds")
                    stage[e, pl.ds(16 * h, 16)] = (
                        bufA[e, pl.ds(16 * h, 16)] * exh)
                stage[e, pl.ds(64, 16)] = t * exv
                return c2

            lax.fori_loop(0, K, edge, 0)
            pltpu.sync_copy(stage, acc.at[dst_v.at[j]], add=True)
            return carry

        lax.fori_loop(0, NCH12, chunk, 0)
        plsc.subcore_barrier()

        def wchunk(i, carry):
            ch = sid + 16 * i

            @pl.when(ch < NRC)
            def _():
                st = pl.multiple_of(ch * K, 8)
                pltpu.sync_copy(acc.at[pl.ds(st, K)],
                                out.at[cid, pl.ds(st, K)])
            return carry

        lax.fori_loop(0, (NRC + 15) // 16, wchunk, 0)

    return pl.kernel(
        body,
        mesh=mesh,
        compiler_params=_SC_PARAMS,
        out_type=jax.ShapeDtypeStruct((2, N, RW12), jnp.float32),
        scratch_types=[
            pltpu.VMEM((NCH12, K), jnp.int32),
            pltpu.VMEM((NCH12, K), jnp.int32),
            pltpu.VMEM((K, RW12), jnp.float32),
            pltpu.VMEM((K, 16), jnp.float32),
            pltpu.VMEM((K, RW12), jnp.float32),
            pltpu.VMEM_SHARED((N, RW12), jnp.float32),
            pltpu.SemaphoreType.DMA,
            pltpu.SemaphoreType.DMA,
        ],
    )


def _make_mp3():
    mesh = plsc.VectorSubcoreMesh(core_axis_name="c", subcore_axis_name="s")

    def body(tableA, tableB, srcr, dstr, out, src_v, dst_v, bufA, bufB,
             stage, acc, semA, semB):
        cid = lax.axis_index("c")
        sid = lax.axis_index("s")
        wid = sid * 2 + cid

        pltpu.sync_copy(srcr.at[wid], src_v)
        pltpu.sync_copy(dstr.at[wid], dst_v)

        zero16 = jnp.zeros((16,), jnp.float32)

        def zrow(r, carry):
            for v in range(3):
                stage[r, pl.ds(16 * v, 16)] = zero16
            return carry

        lax.fori_loop(0, K, zrow, 0)

        def zchunk(i, carry):
            ch = sid + 16 * i

            @pl.when(ch < NRC)
            def _():
                pltpu.sync_copy(stage,
                                acc.at[pl.ds(pl.multiple_of(ch * K, 8), K)])
            return carry

        lax.fori_loop(0, (NRC + 15) // 16, zchunk, 0)
        plsc.subcore_barrier()

        full8 = jnp.full((16,), 8, jnp.int32)
        full0 = jnp.full((16,), 0, jnp.int32)

        def chunk(j, carry):
            cpA = pltpu.async_copy(tableA.at[src_v.at[j]], bufA, semA)
            cpB = pltpu.async_copy(tableB.at[dst_v.at[j]], bufB, semB)
            cpA.wait()
            cpB.wait()

            def edge(e, c2):
                t = bufA[e, pl.ds(32, 16)]
                va = t.at[full8].get(mode="promise_in_bounds")
                vb = bufB[e, pl.ds(0, 16)]
                vb = vb.at[full0].get(mode="promise_in_bounds")
                sv = va + vb
                exv = jnp.exp(jnp.maximum(sv, 0.2 * sv))
                for v in range(2):
                    stage[e, pl.ds(16 * v, 16)] = (
                        bufA[e, pl.ds(16 * v, 16)] * exv)
                stage[e, pl.ds(32, 16)] = t * exv
                return c2

            lax.fori_loop(0, K, edge, 0)
            pltpu.sync_copy(stage, acc.at[dst_v.at[j]], add=True)
            return carry

        lax.fori_loop(0, NCH3, chunk, 0)
        plsc.subcore_barrier()

        def wchunk(i, carry):
            ch = sid + 16 * i

            @pl.when(ch < NRC)
            def _():
                st = pl.multiple_of(ch * K, 8)
                pltpu.sync_copy(acc.at[pl.ds(st, K)],
                                out.at[cid, pl.ds(st, K)])
            return carry

        lax.fori_loop(0, (NRC + 15) // 16, wchunk, 0)

    return pl.kernel(
        body,
        mesh=mesh,
        compiler_params=_SC_PARAMS,
        out_type=jax.ShapeDtypeStruct((2, N, RW3), jnp.float32),
        scratch_types=[
            pltpu.VMEM((NCH3, K), jnp.int32),
            pltpu.VMEM((NCH3, K), jnp.int32),
            pltpu.VMEM((K, RW3), jnp.float32),
            pltpu.VMEM((K, 16), jnp.float32),
            pltpu.VMEM((K, RW3), jnp.float32),
            pltpu.VMEM_SHARED((N, RW3), jnp.float32),
            pltpu.SemaphoreType.DMA,
            pltpu.SemaphoreType.DMA,
        ],
    )


def _emit_tables12(h, asrc, adst, a_ref, b_ref):
    ones8 = jnp.ones((N, 8), jnp.float32)
    zeros4 = jnp.zeros((N, 4), jnp.float32)
    a_ref[0:N, 0:64] = h[:, 0:64]
    a_ref[0:N, 64:72] = ones8
    a_ref[0:N, 72:76] = asrc[:, 0:4]
    a_ref[0:N, 76:80] = zeros4
    a_ref[N:2 * N, 0:64] = h[:, 64:128]
    a_ref[N:2 * N, 64:72] = ones8
    a_ref[N:2 * N, 72:76] = asrc[:, 4:8]
    a_ref[N:2 * N, 76:80] = zeros4
    b_ref[:, 0:8] = adst
    b_ref[:, 8:16] = jnp.zeros((N, 8), jnp.float32)


def _tc_first(x, W, asrc_m, adst_m):
    def f(x_ref, w_ref, as_ref, ad_ref, a_ref, b_ref):
        h = jnp.dot(x_ref[...], w_ref[...], preferred_element_type=jnp.float32)
        asrc = jnp.dot(h, as_ref[...], preferred_element_type=jnp.float32)
        adst = jnp.dot(h, ad_ref[...], preferred_element_type=jnp.float32)
        _emit_tables12(h, asrc, adst, a_ref, b_ref)

    return pl.pallas_call(
        f,
        out_shape=(jax.ShapeDtypeStruct((2 * N, RW12), jnp.float32),
                   jax.ShapeDtypeStruct((N, 16), jnp.float32)),
    )(x, W, asrc_m, adst_m)


def _combine_bn_relu(p_ref, sx_ref, bias_ref, g_ref, be_ref):
    acc = jnp.concatenate([p_ref[0][:, 0:64], p_ref[1][:, 0:64]], axis=1)
    s8 = jnp.concatenate([p_ref[0][:, 64:68], p_ref[1][:, 64:68]], axis=1)
    s = jnp.dot(s8, sx_ref[...], preferred_element_type=jnp.float32)
    g = acc / (s + 1e-16) + bias_ref[...]
    mu = jnp.mean(g, axis=0, keepdims=True)
    var = jnp.mean((g - mu) ** 2, axis=0, keepdims=True)
    y = g_ref[...] * (g - mu) * lax.rsqrt(var + 1e-5) + be_ref[...]
    return jnp.maximum(y, 0.0)


def _tc_mid(part, gamma, beta, bias, W, asrc_m, adst_m, sexp):
    def f(p_ref, sx_ref, bias_ref, g_ref, be_ref, w_ref, as_ref, ad_ref,
          a_ref, b_ref):
        y = _combine_bn_relu(p_ref, sx_ref, bias_ref, g_ref, be_ref)
        h = jnp.dot(y, w_ref[...], preferred_element_type=jnp.float32)
        asrc = jnp.dot(h, as_ref[...], preferred_element_type=jnp.float32)
        adst = jnp.dot(h, ad_ref[...], preferred_element_type=jnp.float32)
        _emit_tables12(h, asrc, adst, a_ref, b_ref)

    return pl.pallas_call(
        f,
        out_shape=(jax.ShapeDtypeStruct((2 * N, RW12), jnp.float32),
                   jax.ShapeDtypeStruct((N, 16), jnp.float32)),
    )(part, sexp, bias, gamma, beta, W, asrc_m, adst_m)


def _tc_mid3(part, gamma, beta, bias, W, as3, ad3, sexp):
    def f(p_ref, sx_ref, bias_ref, g_ref, be_ref, w_ref, as_ref, ad_ref,
          a_ref, b_ref):
        y = _combine_bn_relu(p_ref, sx_ref, bias_ref, g_ref, be_ref)
        h = jnp.dot(y, w_ref[...], preferred_element_type=jnp.float32)
        a_ref[:, 0:40] = h
        a_ref[:, 40:41] = jnp.dot(h, as_ref[...],
                                  preferred_element_type=jnp.float32)
        a_ref[:, 41:42] = jnp.ones((N, 1), jnp.float32)
        a_ref[:, 42:48] = jnp.zeros((N, 6), jnp.float32)
        b_ref[:, 0:1] = jnp.dot(h, ad_ref[...],
                                preferred_element_type=jnp.float32)
        b_ref[:, 1:16] = jnp.zeros((N, 15), jnp.float32)

    return pl.pallas_call(
        f,
        out_shape=(jax.ShapeDtypeStruct((N, RW3), jnp.float32),
                   jax.ShapeDtypeStruct((N, 16), jnp.float32)),
    )(part, sexp, bias, gamma, beta, W, as3, ad3)


def _tc_final(part, bias):
    def f(p_ref, bias_ref, o_ref):
        acc = p_ref[0] + p_ref[1]
        logits = acc[:, 0:40] / (acc[:, 41:42] + 1e-16) + bias_ref[...]
        m = jnp.max(logits, axis=1, keepdims=True)
        z = logits - m
        lse = jnp.log(jnp.sum(jnp.exp(z), axis=1, keepdims=True))
        o_ref[...] = z - lse

    return pl.pallas_call(
        f,
        out_shape=jax.ShapeDtypeStruct((N, 40), jnp.float32),
    )(part, bias)


def kernel(x, edge_index, W1, att_src1, att_dst1, bias1, gamma1, beta1,
           W2, att_src2, att_dst2, bias2, gamma2, beta2,
           W3, att_src3, att_dst3, bias3):
    ei = edge_index.astype(jnp.int32)
    src16 = ei[0].reshape(16, NCH12, K)
    srcr12 = jnp.stack([src16, src16 + N])
    dstr12 = ei[1].reshape(16, NCH12, K)
    srcr3 = ei[0].reshape(32, NCH3, K)
    dstr3 = ei[1].reshape(32, NCH3, K)

    mask8 = jnp.kron(jnp.eye(8, dtype=jnp.float32),
                     jnp.ones((16, 1), jnp.float32))
    sexp = jnp.kron(jnp.eye(8, dtype=jnp.float32),
                    jnp.ones((1, 16), jnp.float32))
    asrc1_m = mask8 * att_src1.reshape(128, 1)
    adst1_m = mask8 * att_dst1.reshape(128, 1)
    asrc2_m = mask8 * att_src2.reshape(128, 1)
    adst2_m = mask8 * att_dst2.reshape(128, 1)
    as3 = att_src3.reshape(40, 1)
    ad3 = att_dst3.reshape(40, 1)

    b1 = bias1.reshape(1, 128)
    g1 = gamma1.reshape(1, 128)
    be1 = beta1.reshape(1, 128)
    b2 = bias2.reshape(1, 128)
    g2 = gamma2.reshape(1, 128)
    be2 = beta2.reshape(1, 128)
    b3 = bias3.reshape(1, 40)

    mp12 = _make_mp12()
    mp3 = _make_mp3()

    tA, tB = _tc_first(x, W1, asrc1_m, adst1_m)
    part = mp12(tA, tB, srcr12, dstr12)
    tA, tB = _tc_mid(part, g1, be1, b1, W2, asrc2_m, adst2_m, sexp)
    part = mp12(tA, tB, srcr12, dstr12)
    tA, tB = _tc_mid3(part, g2, be2, b2, W3, as3, ad3, sexp)
    part3 = mp3(tA, tB, srcr3, dstr3)
    return _tc_final(part3, b3)


# speedup vs baseline: 81.4058x; 1.6301x over previous
import jax
import jax.numpy as jnp
from jax import lax
from jax.experimental import pallas as pl
from jax.experimental.pallas import tpu as pltpu
from jax.experimental.pallas import tpu_sc as plsc

N = 10000
E = 320000
K = 80
NRC = N // K
RW12 = 80
RW3 = 48
NCH12 = E // 16 // K
NCH3 = E // 32 // K

_SC_PARAMS = pltpu.CompilerParams(use_tc_tiling_on_sc=False)


def _make_mp12():
    mesh = plsc.VectorSubcoreMesh(core_axis_name="c", subcore_axis_name="s")

    def body(tableA, tableB, srcr, dstr, out, src_v, dst_v, bufA, bufB,
             stage, bufA2, bufB2, stage2, acc, semG0, semG1, semS0, semS1):
        cid = lax.axis_index("c")
        sid = lax.axis_index("s")

        pltpu.sync_copy(srcr.at[cid, sid], src_v)
        pltpu.sync_copy(dstr.at[sid], dst_v)

        zero16 = jnp.zeros((16,), jnp.float32)

        def zrow(r, carry):
            for v in range(5):
                stage[r, pl.ds(16 * v, 16)] = zero16
            return carry

        lax.fori_loop(0, K, zrow, 0)

        def zchunk(i, carry):
            ch = sid + 16 * i

            @pl.when(ch < NRC)
            def _():
                pltpu.sync_copy(stage,
                                acc.at[pl.ds(pl.multiple_of(ch * K, 8), K)])
            return carry

        lax.fori_loop(0, (NRC + 15) // 16, zchunk, 0)
        plsc.subcore_barrier()

        lane4 = lax.iota(jnp.int32, 16) & 3
        shiftA = lane4 + 8
        shiftB = lane4 + 4 * cid

        def compute(bA, bB, st):
            def edge(e, c2):
                t = bA[e, pl.ds(64, 16)]
                va = t.at[shiftA].get(mode="promise_in_bounds")
                vb = bB[e, pl.ds(0, 16)]
                vb = vb.at[shiftB].get(mode="promise_in_bounds")
                sv = va + vb
                exv = jnp.exp(jnp.maximum(sv, 0.2 * sv))
                for h in range(4):
                    exh = exv.at[jnp.full((16,), h, jnp.int32)].get(
                        mode="promise_in_bounds")
                    st[e, pl.ds(16 * h, 16)] = bA[e, pl.ds(16 * h, 16)] * exh
                st[e, pl.ds(64, 16)] = t * exv
                return c2

            lax.fori_loop(0, K, edge, 0)

        bufs = ((bufA, bufB, stage, semG0, semS0),
                (bufA2, bufB2, stage2, semG1, semS1))

        def start_gather(j, b):
            bA, bB, _, sG, _ = bufs[b]
            pltpu.make_async_copy(tableA.at[src_v.at[j]], bA, sG).start()
            pltpu.make_async_copy(tableB.at[dst_v.at[j]], bB, sG).start()

        def wait_gather(j, b):
            bA, bB, _, sG, _ = bufs[b]
            pltpu.make_async_copy(tableA.at[src_v.at[j]], bA, sG).wait()
            pltpu.make_async_copy(tableB.at[dst_v.at[j]], bB, sG).wait()

        def start_scatter(j, b):
            _, _, st, _, sS = bufs[b]
            pltpu.make_async_copy(st, acc.at[dst_v.at[j]], sS).start(add=True)

        def wait_scatter(j, b):
            _, _, st, _, sS = bufs[b]
            pltpu.make_async_copy(st, acc.at[dst_v.at[j]], sS).wait()

        start_gather(0, 0)

        def pipe(j2, carry):
            for b in range(2):
                j = 2 * j2 + b

                @pl.when(j + 1 < NCH12)
                def _():
                    start_gather(j + 1, 1 - b)
                wait_gather(j, b)

                @pl.when(j2 > 0)
                def _():
                    wait_scatter(j, b)
                bA, bB, st, _, _ = bufs[b]
                compute(bA, bB, st)
                start_scatter(j, b)
            return carry

        lax.fori_loop(0, NCH12 // 2, pipe, 0)
        wait_scatter(NCH12 - 2, 0)
        wait_scatter(NCH12 - 1, 1)
        plsc.subcore_barrier()

        def wchunk(i, carry):
            ch = sid + 16 * i

            @pl.when(ch < NRC)
            def _():
                st = pl.multiple_of(ch * K, 8)
                pltpu.sync_copy(acc.at[pl.ds(st, K)],
                                out.at[cid, pl.ds(st, K)])
            return carry

        lax.fori_loop(0, (NRC + 15) // 16, wchunk, 0)

    return pl.kernel(
        body,
        mesh=mesh,
        compiler_params=_SC_PARAMS,
        out_type=jax.ShapeDtypeStruct((2, N, RW12), jnp.float32),
        scratch_types=[
            pltpu.VMEM((NCH12, K), jnp.int32),
            pltpu.VMEM((NCH12, K), jnp.int32),
            pltpu.VMEM((K, RW12), jnp.float32),
            pltpu.VMEM((K, 16), jnp.float32),
            pltpu.VMEM((K, RW12), jnp.float32),
            pltpu.VMEM((K, RW12), jnp.float32),
            pltpu.VMEM((K, 16), jnp.float32),
            pltpu.VMEM((K, RW12), jnp.float32),
            pltpu.VMEM_SHARED((N, RW12), jnp.float32),
            pltpu.SemaphoreType.DMA,
            pltpu.SemaphoreType.DMA,
            pltpu.SemaphoreType.DMA,
            pltpu.SemaphoreType.DMA,
        ],
    )


def _make_mp3():
    mesh = plsc.VectorSubcoreMesh(core_axis_name="c", subcore_axis_name="s")

    def body(tableA, tableB, srcr, dstr, out, src_v, dst_v, bufA, bufB,
             stage, bufA2, bufB2, stage2, acc, semG0, semG1, semS0, semS1):
        cid = lax.axis_index("c")
        sid = lax.axis_index("s")
        wid = sid * 2 + cid

        pltpu.sync_copy(srcr.at[wid], src_v)
        pltpu.sync_copy(dstr.at[wid], dst_v)

        zero16 = jnp.zeros((16,), jnp.float32)

        def zrow(r, carry):
            for v in range(3):
                stage[r, pl.ds(16 * v, 16)] = zero16
            return carry

        lax.fori_loop(0, K, zrow, 0)

        def zchunk(i, carry):
            ch = sid + 16 * i

            @pl.when(ch < NRC)
            def _():
                pltpu.sync_copy(stage,
                                acc.at[pl.ds(pl.multiple_of(ch * K, 8), K)])
            return carry

        lax.fori_loop(0, (NRC + 15) // 16, zchunk, 0)
        plsc.subcore_barrier()

        full8 = jnp.full((16,), 8, jnp.int32)
        full0 = jnp.full((16,), 0, jnp.int32)

        def compute(bA, bB, st):
            def edge(e, c2):
                t = bA[e, pl.ds(32, 16)]
                va = t.at[full8].get(mode="promise_in_bounds")
                vb = bB[e, pl.ds(0, 16)]
                vb = vb.at[full0].get(mode="promise_in_bounds")
                sv = va + vb
                exv = jnp.exp(jnp.maximum(sv, 0.2 * sv))
                for v in range(2):
                    st[e, pl.ds(16 * v, 16)] = bA[e, pl.ds(16 * v, 16)] * exv
                st[e, pl.ds(32, 16)] = t * exv
                return c2

            lax.fori_loop(0, K, edge, 0)

        bufs = ((bufA, bufB, stage, semG0, semS0),
                (bufA2, bufB2, stage2, semG1, semS1))

        def start_gather(j, b):
            bA, bB, _, sG, _ = bufs[b]
            pltpu.make_async_copy(tableA.at[src_v.at[j]], bA, sG).start()
            pltpu.make_async_copy(tableB.at[dst_v.at[j]], bB, sG).start()

        def wait_gather(j, b):
            bA, bB, _, sG, _ = bufs[b]
            pltpu.make_async_copy(tableA.at[src_v.at[j]], bA, sG).wait()
            pltpu.make_async_copy(tableB.at[dst_v.at[j]], bB, sG).wait()

        def start_scatter(j, b):
            _, _, st, _, sS = bufs[b]
            pltpu.make_async_copy(st, acc.at[dst_v.at[j]], sS).start(add=True)

        def wait_scatter(j, b):
            _, _, st, _, sS = bufs[b]
            pltpu.make_async_copy(st, acc.at[dst_v.at[j]], sS).wait()

        start_gather(0, 0)

        def pipe(j2, carry):
            for b in range(2):
                j = 2 * j2 + b

                @pl.when(j + 1 < NCH3)
                def _():
                    start_gather(j + 1, 1 - b)
                wait_gather(j, b)

                @pl.when(j2 > 0)
                def _():
                    wait_scatter(j, b)
                bA, bB, st, _, _ = bufs[b]
                compute(bA, bB, st)
                start_scatter(j, b)
            return carry

        lax.fori_loop(0, NCH3 // 2, pipe, 0)
        jt = NCH3 - 1
        wait_gather(jt, 0)
        wait_scatter(jt, 0)
        compute(bufA, bufB, stage)
        start_scatter(jt, 0)
        wait_scatter(jt, 0)
        wait_scatter(jt - 1, 1)
        plsc.subcore_barrier()

        def wchunk(i, carry):
            ch = sid + 16 * i

            @pl.when(ch < NRC)
            def _():
                st = pl.multiple_of(ch * K, 8)
                pltpu.sync_copy(acc.at[pl.ds(st, K)],
                                out.at[cid, pl.ds(st, K)])
            return carry

        lax.fori_loop(0, (NRC + 15) // 16, wchunk, 0)

    return pl.kernel(
        body,
        mesh=mesh,
        compiler_params=_SC_PARAMS,
        out_type=jax.ShapeDtypeStruct((2, N, RW3), jnp.float32),
        scratch_types=[
            pltpu.VMEM((NCH3, K), jnp.int32),
            pltpu.VMEM((NCH3, K), jnp.int32),
            pltpu.VMEM((K, RW3), jnp.float32),
            pltpu.VMEM((K, 16), jnp.float32),
            pltpu.VMEM((K, RW3), jnp.float32),
            pltpu.VMEM((K, RW3), jnp.float32),
            pltpu.VMEM((K, 16), jnp.float32),
            pltpu.VMEM((K, RW3), jnp.float32),
            pltpu.VMEM_SHARED((N, RW3), jnp.float32),
            pltpu.SemaphoreType.DMA,
            pltpu.SemaphoreType.DMA,
            pltpu.SemaphoreType.DMA,
            pltpu.SemaphoreType.DMA,
        ],
    )


def _emit_tables12(h, asrc, adst, a_ref, b_ref):
    ones8 = jnp.ones((N, 8), jnp.float32)
    zeros4 = jnp.zeros((N, 4), jnp.float32)
    a_ref[0:N, 0:64] = h[:, 0:64]
    a_ref[0:N, 64:72] = ones8
    a_ref[0:N, 72:76] = asrc[:, 0:4]
    a_ref[0:N, 76:80] = zeros4
    a_ref[N:2 * N, 0:64] = h[:, 64:128]
    a_ref[N:2 * N, 64:72] = ones8
    a_ref[N:2 * N, 72:76] = asrc[:, 4:8]
    a_ref[N:2 * N, 76:80] = zeros4
    b_ref[:, 0:8] = adst
    b_ref[:, 8:16] = jnp.zeros((N, 8), jnp.float32)


def _tc_first(x, W, asrc_m, adst_m):
    def f(x_ref, w_ref, as_ref, ad_ref, a_ref, b_ref):
        h = jnp.dot(x_ref[...], w_ref[...], preferred_element_type=jnp.float32)
        asrc = jnp.dot(h, as_ref[...], preferred_element_type=jnp.float32)
        adst = jnp.dot(h, ad_ref[...], preferred_element_type=jnp.float32)
        _emit_tables12(h, asrc, adst, a_ref, b_ref)

    return pl.pallas_call(
        f,
        out_shape=(jax.ShapeDtypeStruct((2 * N, RW12), jnp.float32),
                   jax.ShapeDtypeStruct((N, 16), jnp.float32)),
    )(x, W, asrc_m, adst_m)


def _combine_bn_relu(p_ref, sx_ref, bias_ref, g_ref, be_ref):
    acc = jnp.concatenate([p_ref[0][:, 0:64], p_ref[1][:, 0:64]], axis=1)
    s8 = jnp.concatenate([p_ref[0][:, 64:68], p_ref[1][:, 64:68]], axis=1)
    s = jnp.dot(s8, sx_ref[...], preferred_element_type=jnp.float32)
    g = acc / (s + 1e-16) + bias_ref[...]
    mu = jnp.mean(g, axis=0, keepdims=True)
    var = jnp.mean((g - mu) ** 2, axis=0, keepdims=True)
    y = g_ref[...] * (g - mu) * lax.rsqrt(var + 1e-5) + be_ref[...]
    return jnp.maximum(y, 0.0)


def _tc_mid(part, gamma, beta, bias, W, asrc_m, adst_m, sexp):
    def f(p_ref, sx_ref, bias_ref, g_ref, be_ref, w_ref, as_ref, ad_ref,
          a_ref, b_ref):
        y = _combine_bn_relu(p_ref, sx_ref, bias_ref, g_ref, be_ref)
        h = jnp.dot(y, w_ref[...], preferred_element_type=jnp.float32)
        asrc = jnp.dot(h, as_ref[...], preferred_element_type=jnp.float32)
        adst = jnp.dot(h, ad_ref[...], preferred_element_type=jnp.float32)
        _emit_tables12(h, asrc, adst, a_ref, b_ref)

    return pl.pallas_call(
        f,
        out_shape=(jax.ShapeDtypeStruct((2 * N, RW12), jnp.float32),
                   jax.ShapeDtypeStruct((N, 16), jnp.float32)),
    )(part, sexp, bias, gamma, beta, W, asrc_m, adst_m)


def _tc_mid3(part, gamma, beta, bias, W, as3, ad3, sexp):
    def f(p_ref, sx_ref, bias_ref, g_ref, be_ref, w_ref, as_ref, ad_ref,
          a_ref, b_ref):
        y = _combine_bn_relu(p_ref, sx_ref, bias_ref, g_ref, be_ref)
        h = jnp.dot(y, w_ref[...], preferred_element_type=jnp.float32)
        a_ref[:, 0:40] = h
        a_ref[:, 40:41] = jnp.dot(h, as_ref[...],
                                  preferred_element_type=jnp.float32)
        a_ref[:, 41:42] = jnp.ones((N, 1), jnp.float32)
        a_ref[:, 42:48] = jnp.zeros((N, 6), jnp.float32)
        b_ref[:, 0:1] = jnp.dot(h, ad_ref[...],
                                preferred_element_type=jnp.float32)
        b_ref[:, 1:16] = jnp.zeros((N, 15), jnp.float32)

    return pl.pallas_call(
        f,
        out_shape=(jax.ShapeDtypeStruct((N, RW3), jnp.float32),
                   jax.ShapeDtypeStruct((N, 16), jnp.float32)),
    )(part, sexp, bias, gamma, beta, W, as3, ad3)


def _tc_final(part, bias):
    def f(p_ref, bias_ref, o_ref):
        acc = p_ref[0] + p_ref[1]
        logits = acc[:, 0:40] / (acc[:, 41:42] + 1e-16) + bias_ref[...]
        m = jnp.max(logits, axis=1, keepdims=True)
        z = logits - m
        lse = jnp.log(jnp.sum(jnp.exp(z), axis=1, keepdims=True))
        o_ref[...] = z - lse

    return pl.pallas_call(
        f,
        out_shape=jax.ShapeDtypeStruct((N, 40), jnp.float32),
    )(part, bias)


def kernel(x, edge_index, W1, att_src1, att_dst1, bias1, gamma1, beta1,
           W2, att_src2, att_dst2, bias2, gamma2, beta2,
           W3, att_src3, att_dst3, bias3):
    ei = edge_index.astype(jnp.int32)
    src16 = ei[0].reshape(16, NCH12, K)
    srcr12 = jnp.stack([src16, src16 + N])
    dstr12 = ei[1].reshape(16, NCH12, K)
    srcr3 = ei[0].reshape(32, NCH3, K)
    dstr3 = ei[1].reshape(32, NCH3, K)

    mask8 = jnp.kron(jnp.eye(8, dtype=jnp.float32),
                     jnp.ones((16, 1), jnp.float32))
    sexp = jnp.kron(jnp.eye(8, dtype=jnp.float32),
                    jnp.ones((1, 16), jnp.float32))
    asrc1_m = mask8 * att_src1.reshape(128, 1)
    adst1_m = mask8 * att_dst1.reshape(128, 1)
    asrc2_m = mask8 * att_src2.reshape(128, 1)
    adst2_m = mask8 * att_dst2.reshape(128, 1)
    as3 = att_src3.reshape(40, 1)
    ad3 = att_dst3.reshape(40, 1)

    b1 = bias1.reshape(1, 128)
    g1 = gamma1.reshape(1, 128)
    be1 = beta1.reshape(1, 128)
    b2 = bias2.reshape(1, 128)
    g2 = gamma2.reshape(1, 128)
    be2 = beta2.reshape(1, 128)
    b3 = bias3.reshape(1, 40)

    mp12 = _make_mp12()
    mp3 = _make_mp3()

    tA, tB = _tc_first(x, W1, asrc1_m, adst1_m)
    part = mp12(tA, tB, srcr12, dstr12)
    tA, tB = _tc_mid(part, g1, be1, b1, W2, asrc2_m, adst2_m, sexp)
    part = mp12(tA, tB, srcr12, dstr12)
    tA, tB = _tc_mid3(part, g2, be2, b2, W3, as3, ad3, sexp)
    part3 = mp3(tA, tB, srcr3, dstr3)
    return _tc_final(part3, b3)


# speedup vs baseline: 82.5244x; 1.0137x over previous
import jax
import jax.numpy as jnp
from jax import lax
from jax.experimental import pallas as pl
from jax.experimental.pallas import tpu as pltpu
from jax.experimental.pallas import tpu_sc as plsc

N = 10000
E = 320000
K = 80
NRC = N // K
RW12 = 80
RW3 = 48
NCH12 = E // 16 // K
NCH3 = E // 32 // K

_SC_PARAMS = pltpu.CompilerParams(use_tc_tiling_on_sc=False)


def _make_mp12():
    mesh = plsc.VectorSubcoreMesh(core_axis_name="c", subcore_axis_name="s")

    def body(tableA, tableB, srcr, dstr, out, src_v, dst_v, bufA, bufB,
             stage, bufA2, bufB2, stage2, acc,
             semG0, semG1, semS0, semS1):
        cid = lax.axis_index("c")
        sid = lax.axis_index("s")

        pltpu.sync_copy(srcr.at[cid, sid], src_v)
        pltpu.sync_copy(dstr.at[sid], dst_v)

        zero16 = jnp.zeros((16,), jnp.float32)

        def zrow(r, carry):
            for v in range(5):
                stage[r, pl.ds(16 * v, 16)] = zero16
            return carry

        lax.fori_loop(0, K, zrow, 0)

        def zchunk(i, carry):
            ch = sid + 16 * i

            @pl.when(ch < NRC)
            def _():
                pltpu.sync_copy(stage,
                                acc.at[pl.ds(pl.multiple_of(ch * K, 8), K)])
            return carry

        lax.fori_loop(0, (NRC + 15) // 16, zchunk, 0)
        plsc.subcore_barrier()

        lane4 = lax.iota(jnp.int32, 16) & 3
        shiftA = lane4 + 8
        shiftB = lane4 + 4 * cid
        full_h = [jnp.full((16,), h, jnp.int32) for h in range(4)]

        def compute(bA, bB, st):
            def edge4(q, c2):
                for u in range(4):
                    e = q * 4 + u
                    t = bA[e, pl.ds(64, 16)]
                    va = t.at[shiftA].get(mode="promise_in_bounds")
                    vb = bB[e, pl.ds(0, 16)]
                    vb = vb.at[shiftB].get(mode="promise_in_bounds")
                    sv = va + vb
                    exv = jnp.exp(jnp.maximum(sv, 0.2 * sv))
                    for h in range(4):
                        exh = exv.at[full_h[h]].get(
                            mode="promise_in_bounds")
                        st[e, pl.ds(16 * h, 16)] = (
                            bA[e, pl.ds(16 * h, 16)] * exh)
                    st[e, pl.ds(64, 16)] = exv
                return c2

            lax.fori_loop(0, K // 4, edge4, 0)

        bufs = ((bufA, bufB, stage, semG0, semS0),
                (bufA2, bufB2, stage2, semG1, semS1))

        def start_gather(j, b):
            bA, bB, _, sG, _ = bufs[b]
            pltpu.make_async_copy(tableA.at[src_v.at[j]], bA, sG).start()
            pltpu.make_async_copy(tableB.at[dst_v.at[j]], bB, sG).start()

        def wait_gather(j, b):
            bA, bB, _, sG, _ = bufs[b]
            pltpu.make_async_copy(tableA.at[src_v.at[j]], bA, sG).wait()
            pltpu.make_async_copy(tableB.at[dst_v.at[j]], bB, sG).wait()

        def start_scatter(j, b):
            _, _, st, _, sS = bufs[b]
            pltpu.make_async_copy(st, acc.at[dst_v.at[j]], sS).start(add=True)

        def wait_scatter(j, b):
            _, _, st, _, sS = bufs[b]
            pltpu.make_async_copy(st, acc.at[dst_v.at[j]], sS).wait()

        start_gather(0, 0)

        def pipe(j2, carry):
            for b in range(2):
                j = 2 * j2 + b

                @pl.when(j + 1 < NCH12)
                def _():
                    start_gather(j + 1, 1 - b)
                wait_gather(j, b)

                @pl.when(j2 > 0)
                def _():
                    wait_scatter(j, b)
                bA, bB, st, _, _ = bufs[b]
                compute(bA, bB, st)
                start_scatter(j, b)
            return carry

        lax.fori_loop(0, NCH12 // 2, pipe, 0)
        wait_scatter(NCH12 - 2, 0)
        wait_scatter(NCH12 - 1, 1)
        plsc.subcore_barrier()

        def wchunk(i, carry):
            ch = sid + 16 * i

            @pl.when(ch < NRC)
            def _():
                st = pl.multiple_of(ch * K, 8)
                pltpu.sync_copy(acc.at[pl.ds(st, K)],
                                out.at[cid, pl.ds(st, K)])
            return carry

        lax.fori_loop(0, (NRC + 15) // 16, wchunk, 0)

    return pl.kernel(
        body,
        mesh=mesh,
        compiler_params=_SC_PARAMS,
        out_type=jax.ShapeDtypeStruct((2, N, RW12), jnp.float32),
        scratch_types=[
            pltpu.VMEM((NCH12, K), jnp.int32),
            pltpu.VMEM((NCH12, K), jnp.int32),
            pltpu.VMEM((K, RW12), jnp.float32),
            pltpu.VMEM((K, 16), jnp.float32),
            pltpu.VMEM((K, RW12), jnp.float32),
            pltpu.VMEM((K, RW12), jnp.float32),
            pltpu.VMEM((K, 16), jnp.float32),
            pltpu.VMEM((K, RW12), jnp.float32),
            pltpu.VMEM_SHARED((N, RW12), jnp.float32),
            pltpu.SemaphoreType.DMA,
            pltpu.SemaphoreType.DMA,
            pltpu.SemaphoreType.DMA,
            pltpu.SemaphoreType.DMA,
        ],
    )


def _make_mp3():
    mesh = plsc.VectorSubcoreMesh(core_axis_name="c", subcore_axis_name="s")

    def body(tableA, tableB, srcr, dstr, out, src_v, dst_v, bufA, bufB,
             stage, bufA2, bufB2, stage2, acc,
             semG0, semG1, semS0, semS1):
        cid = lax.axis_index("c")
        sid = lax.axis_index("s")
        wid = sid * 2 + cid

        pltpu.sync_copy(srcr.at[wid], src_v)
        pltpu.sync_copy(dstr.at[wid], dst_v)

        zero16 = jnp.zeros((16,), jnp.float32)

        def zrow(r, carry):
            for v in range(3):
                stage[r, pl.ds(16 * v, 16)] = zero16
            return carry

        lax.fori_loop(0, K, zrow, 0)

        def zchunk(i, carry):
            ch = sid + 16 * i

            @pl.when(ch < NRC)
            def _():
                pltpu.sync_copy(stage,
                                acc.at[pl.ds(pl.multiple_of(ch * K, 8), K)])
            return carry

        lax.fori_loop(0, (NRC + 15) // 16, zchunk, 0)
        plsc.subcore_barrier()

        full8 = jnp.full((16,), 8, jnp.int32)
        full0 = jnp.full((16,), 0, jnp.int32)

        def compute(bA, bB, st):
            def edge4(q, c2):
                for u in range(4):
                    e = q * 4 + u
                    t = bA[e, pl.ds(32, 16)]
                    va = t.at[full8].get(mode="promise_in_bounds")
                    vb = bB[e, pl.ds(0, 16)]
                    vb = vb.at[full0].get(mode="promise_in_bounds")
                    sv = va + vb
                    exv = jnp.exp(jnp.maximum(sv, 0.2 * sv))
                    for v in range(2):
                        st[e, pl.ds(16 * v, 16)] = (
                            bA[e, pl.ds(16 * v, 16)] * exv)
                    st[e, pl.ds(32, 16)] = t * exv
                return c2

            lax.fori_loop(0, K // 4, edge4, 0)

        bufs = ((bufA, bufB, stage, semG0, semS0),
                (bufA2, bufB2, stage2, semG1, semS1))

        def start_gather(j, b):
            bA, bB, _, sG, _ = bufs[b]
            pltpu.make_async_copy(tableA.at[src_v.at[j]], bA, sG).start()
            pltpu.make_async_copy(tableB.at[dst_v.at[j]], bB, sG).start()

        def wait_gather(j, b):
            bA, bB, _, sG, _ = bufs[b]
            pltpu.make_async_copy(tableA.at[src_v.at[j]], bA, sG).wait()
            pltpu.make_async_copy(tableB.at[dst_v.at[j]], bB, sG).wait()

        def start_scatter(j, b):
            _, _, st, _, sS = bufs[b]
            pltpu.make_async_copy(st, acc.at[dst_v.at[j]], sS).start(add=True)

        def wait_scatter(j, b):
            _, _, st, _, sS = bufs[b]
            pltpu.make_async_copy(st, acc.at[dst_v.at[j]], sS).wait()

        start_gather(0, 0)

        def pipe(j2, carry):
            for b in range(2):
                j = 2 * j2 + b

                @pl.when(j + 1 < NCH3)
                def _():
                    start_gather(j + 1, 1 - b)
                wait_gather(j, b)

                @pl.when(j2 > 0)
                def _():
                    wait_scatter(j, b)
                bA, bB, st, _, _ = bufs[b]
                compute(bA, bB, st)
                start_scatter(j, b)
            return carry

        lax.fori_loop(0, NCH3 // 2, pipe, 0)
        jt = NCH3 - 1
        wait_gather(jt, 0)
        wait_scatter(jt, 0)
        compute(bufA, bufB, stage)
        start_scatter(jt, 0)
        wait_scatter(jt, 0)
        wait_scatter(jt - 1, 1)
        plsc.subcore_barrier()

        def wchunk(i, carry):
            ch = sid + 16 * i

            @pl.when(ch < NRC)
            def _():
                st = pl.multiple_of(ch * K, 8)
                pltpu.sync_copy(acc.at[pl.ds(st, K)],
                                out.at[cid, pl.ds(st, K)])
            return carry

        lax.fori_loop(0, (NRC + 15) // 16, wchunk, 0)

    return pl.kernel(
        body,
        mesh=mesh,
        compiler_params=_SC_PARAMS,
        out_type=jax.ShapeDtypeStruct((2, N, RW3), jnp.float32),
        scratch_types=[
            pltpu.VMEM((NCH3, K), jnp.int32),
            pltpu.VMEM((NCH3, K), jnp.int32),
            pltpu.VMEM((K, RW3), jnp.float32),
            pltpu.VMEM((K, 16), jnp.float32),
            pltpu.VMEM((K, RW3), jnp.float32),
            pltpu.VMEM((K, RW3), jnp.float32),
            pltpu.VMEM((K, 16), jnp.float32),
            pltpu.VMEM((K, RW3), jnp.float32),
            pltpu.VMEM_SHARED((N, RW3), jnp.float32),
            pltpu.SemaphoreType.DMA,
            pltpu.SemaphoreType.DMA,
            pltpu.SemaphoreType.DMA,
            pltpu.SemaphoreType.DMA,
        ],
    )


def _emit_tables12(h, asrc, adst, a_ref, b_ref):
    ones8 = jnp.ones((N, 8), jnp.float32)
    zeros4 = jnp.zeros((N, 4), jnp.float32)
    a_ref[0:N, 0:64] = h[:, 0:64]
    a_ref[0:N, 64:72] = ones8
    a_ref[0:N, 72:76] = asrc[:, 0:4]
    a_ref[0:N, 76:80] = zeros4
    a_ref[N:2 * N, 0:64] = h[:, 64:128]
    a_ref[N:2 * N, 64:72] = ones8
    a_ref[N:2 * N, 72:76] = asrc[:, 4:8]
    a_ref[N:2 * N, 76:80] = zeros4
    b_ref[:, 0:8] = adst
    b_ref[:, 8:16] = jnp.zeros((N, 8), jnp.float32)


def _tc_first(x, W, asrc_m, adst_m):
    def f(x_ref, w_ref, as_ref, ad_ref, a_ref, b_ref):
        h = jnp.dot(x_ref[...], w_ref[...], preferred_element_type=jnp.float32)
        asrc = jnp.dot(h, as_ref[...], preferred_element_type=jnp.float32)
        adst = jnp.dot(h, ad_ref[...], preferred_element_type=jnp.float32)
        _emit_tables12(h, asrc, adst, a_ref, b_ref)

    return pl.pallas_call(
        f,
        out_shape=(jax.ShapeDtypeStruct((2 * N, RW12), jnp.float32),
                   jax.ShapeDtypeStruct((N, 16), jnp.float32)),
    )(x, W, asrc_m, adst_m)


def _combine_bn_relu(p_ref, sx_ref, bias_ref, g_ref, be_ref):
    acc = jnp.concatenate([p_ref[0][:, 0:64], p_ref[1][:, 0:64]], axis=1)
    s8 = jnp.concatenate([p_ref[0][:, 64:68], p_ref[1][:, 64:68]], axis=1)
    s = jnp.dot(s8, sx_ref[...], preferred_element_type=jnp.float32)
    g = acc / (s + 1e-16) + bias_ref[...]
    mu = jnp.mean(g, axis=0, keepdims=True)
    var = jnp.mean((g - mu) ** 2, axis=0, keepdims=True)
    y = g_ref[...] * (g - mu) * lax.rsqrt(var + 1e-5) + be_ref[...]
    return jnp.maximum(y, 0.0)


def _tc_mid(part, gamma, beta, bias, W, asrc_m, adst_m, sexp):
    def f(p_ref, sx_ref, bias_ref, g_ref, be_ref, w_ref, as_ref, ad_ref,
          a_ref, b_ref):
        y = _combine_bn_relu(p_ref, sx_ref, bias_ref, g_ref, be_ref)
        h = jnp.dot(y, w_ref[...], preferred_element_type=jnp.float32)
        asrc = jnp.dot(h, as_ref[...], preferred_element_type=jnp.float32)
        adst = jnp.dot(h, ad_ref[...], preferred_element_type=jnp.float32)
        _emit_tables12(h, asrc, adst, a_ref, b_ref)

    return pl.pallas_call(
        f,
        out_shape=(jax.ShapeDtypeStruct((2 * N, RW12), jnp.float32),
                   jax.ShapeDtypeStruct((N, 16), jnp.float32)),
    )(part, sexp, bias, gamma, beta, W, asrc_m, adst_m)


def _tc_mid3(part, gamma, beta, bias, W, as3, ad3, sexp):
    def f(p_ref, sx_ref, bias_ref, g_ref, be_ref, w_ref, as_ref, ad_ref,
          a_ref, b_ref):
        y = _combine_bn_relu(p_ref, sx_ref, bias_ref, g_ref, be_ref)
        h = jnp.dot(y, w_ref[...], preferred_element_type=jnp.float32)
        a_ref[:, 0:40] = h
        a_ref[:, 40:41] = jnp.dot(h, as_ref[...],
                                  preferred_element_type=jnp.float32)
        a_ref[:, 41:42] = jnp.ones((N, 1), jnp.float32)
        a_ref[:, 42:48] = jnp.zeros((N, 6), jnp.float32)
        b_ref[:, 0:1] = jnp.dot(h, ad_ref[...],
                                preferred_element_type=jnp.float32)
        b_ref[:, 1:16] = jnp.zeros((N, 15), jnp.float32)

    return pl.pallas_call(
        f,
        out_shape=(jax.ShapeDtypeStruct((N, RW3), jnp.float32),
                   jax.ShapeDtypeStruct((N, 16), jnp.float32)),
    )(part, sexp, bias, gamma, beta, W, as3, ad3)


def _tc_final(part, bias):
    def f(p_ref, bias_ref, o_ref):
        acc = p_ref[0] + p_ref[1]
        logits = acc[:, 0:40] / (acc[:, 41:42] + 1e-16) + bias_ref[...]
        m = jnp.max(logits, axis=1, keepdims=True)
        z = logits - m
        lse = jnp.log(jnp.sum(jnp.exp(z), axis=1, keepdims=True))
        o_ref[...] = z - lse

    return pl.pallas_call(
        f,
        out_shape=jax.ShapeDtypeStruct((N, 40), jnp.float32),
    )(part, bias)


def kernel(x, edge_index, W1, att_src1, att_dst1, bias1, gamma1, beta1,
           W2, att_src2, att_dst2, bias2, gamma2, beta2,
           W3, att_src3, att_dst3, bias3):
    ei = edge_index.astype(jnp.int32)
    src16 = ei[0].reshape(16, NCH12, K)
    srcr12 = jnp.stack([src16, src16 + N])
    dstr12 = ei[1].reshape(16, NCH12, K)
    srcr3 = ei[0].reshape(32, NCH3, K)
    dstr3 = ei[1].reshape(32, NCH3, K)

    mask8 = jnp.kron(jnp.eye(8, dtype=jnp.float32),
                     jnp.ones((16, 1), jnp.float32))
    sexp = jnp.kron(jnp.eye(8, dtype=jnp.float32),
                    jnp.ones((1, 16), jnp.float32))
    asrc1_m = mask8 * att_src1.reshape(128, 1)
    adst1_m = mask8 * att_dst1.reshape(128, 1)
    asrc2_m = mask8 * att_src2.reshape(128, 1)
    adst2_m = mask8 * att_dst2.reshape(128, 1)
    as3 = att_src3.reshape(40, 1)
    ad3 = att_dst3.reshape(40, 1)

    b1 = bias1.reshape(1, 128)
    g1 = gamma1.reshape(1, 128)
    be1 = beta1.reshape(1, 128)
    b2 = bias2.reshape(1, 128)
    g2 = gamma2.reshape(1, 128)
    be2 = beta2.reshape(1, 128)
    b3 = bias3.reshape(1, 40)

    mp12 = _make_mp12()
    mp3 = _make_mp3()

    tA, tB = _tc_first(x, W1, asrc1_m, adst1_m)
    part = mp12(tA, tB, srcr12, dstr12)
    tA, tB = _tc_mid(part, g1, be1, b1, W2, asrc2_m, adst2_m, sexp)
    part = mp12(tA, tB, srcr12, dstr12)
    tA, tB = _tc_mid3(part, g2, be2, b2, W3, as3, ad3, sexp)
    part3 = mp3(tA, tB, srcr3, dstr3)
    return _tc_final(part3, b3)
